# baseline (speedup 1.0000x reference)
import functools

import jax
import jax.numpy as jnp
from jax import lax
from jax.experimental import pallas as pl
from jax.experimental.pallas import tpu as pltpu
from jax.experimental.pallas import tpu_sc as plsc

N = 10000
E = 160000
D = 256
H = 128
NG = 128
NC = 2
NT = 16
NPAD = 10240
EPAD = 163840
BN = 1000
F32 = jnp.float32

@functools.cache
def _mesh():
    return plsc.VectorSubcoreMesh(core_axis_name="c", subcore_axis_name="s")


def _per_tile_slabs(sid, fn):
    @pl.when(sid < NT - 1)
    def _():
        fn(sid * 640, 640)

    @pl.when(sid == NT - 1)
    def _():
        fn(9600, 400)


@functools.partial(jax.named_call, name="sc_deg")
def _sc_deg(dstd, zeros128, ones128):
    @functools.partial(
        pl.kernel,
        out_type=jax.ShapeDtypeStruct((NC, N, 128), F32),
        mesh=_mesh(),
        scratch_types=[
            pltpu.VMEM((40, 128), jnp.int32),
            pltpu.VMEM((128, 128), F32),
            pltpu.VMEM_SHARED((NPAD, 128), F32),
        ],
    )
    def k(dst_ref, z_ref, one_ref, out_ref, idxb, ones_v, acc):
        cid = lax.axis_index("c")
        sid = lax.axis_index("s")
        pltpu.sync_copy(z_ref.at[pl.ds(sid * 640, 640)],
                        acc.at[pl.ds(sid * 640, 640)])
        pltpu.sync_copy(dst_ref.at[cid].at[sid], idxb)
        pltpu.sync_copy(one_ref, ones_v)
        plsc.subcore_barrier()

        @pl.loop(0, 40)
        def _(j):
            pltpu.sync_copy(ones_v, acc.at[idxb.at[j]], add=True)

        plsc.subcore_barrier()
        _per_tile_slabs(sid, lambda base, n: pltpu.sync_copy(
            acc.at[pl.ds(base, n)], out_ref.at[cid].at[pl.ds(base, n)]))

    return k(dstd, zeros128, ones128)


@functools.partial(jax.named_call, name="sc_agg")
def _sc_agg(u, srcr, dstr):
    @functools.partial(
        pl.kernel,
        out_type=jax.ShapeDtypeStruct((NC, N, H), F32),
        mesh=_mesh(),
        scratch_types=[
            pltpu.VMEM((40, 128), jnp.int32),
            pltpu.VMEM((40, 128), jnp.int32),
            pltpu.VMEM((128, H), F32),
            pltpu.VMEM((128, H), F32),
            pltpu.VMEM_SHARED((NPAD, H), F32),
            pltpu.SemaphoreType.DMA,
            pltpu.SemaphoreType.DMA,
            pltpu.SemaphoreType.DMA,
            pltpu.SemaphoreType.DMA,
        ],
    )
    def k(u_ref, src_ref, dst_ref, out_ref, sbuf, dbuf, rows_a, rows_b,
          acc, sem_a, sem_b, sem_sa, sem_sb):
        cid = lax.axis_index("c")
        sid = lax.axis_index("s")
        _per_tile_slabs(sid, lambda base, n: pltpu.sync_copy(
            u_ref.at[cid].at[pl.ds(base, n)], acc.at[pl.ds(base, n)]))
        plsc.subcore_barrier()

        def gather(j, rows, sem):
            return pltpu.async_copy(u_ref.at[cid].at[sbuf.at[j]], rows, sem)

        def wait_g(j, rows, sem):
            pltpu.make_async_copy(u_ref.at[cid].at[sbuf.at[j]],
                                  rows, sem).wait()

        def scat(j, rows, sem):
            pltpu.async_copy(rows, acc.at[dbuf.at[j]], sem, add=True)

        def wait_s(j, rows, sem):
            pltpu.make_async_copy(rows, acc.at[dbuf.at[j]], sem).wait()

        for p in range(2):
            pltpu.sync_copy(src_ref.at[sid].at[pl.ds(p * 40, 40)], sbuf)
            pltpu.sync_copy(dst_ref.at[sid].at[pl.ds(p * 40, 40)], dbuf)
            gather(0, rows_a, sem_a)
            gather(1, rows_b, sem_b)

            @pl.loop(0, 40, step=2)
            def _(j):
                wait_g(j, rows_a, sem_a)
                scat(j, rows_a, sem_sa)
                wait_g(j + 1, rows_b, sem_b)
                scat(j + 1, rows_b, sem_sb)

                @pl.when(j + 2 < 40)
                def _():
                    wait_s(j, rows_a, sem_sa)
                    gather(j + 2, rows_a, sem_a)

                @pl.when(j + 3 < 40)
                def _():
                    wait_s(j + 1, rows_b, sem_sb)
                    gather(j + 3, rows_b, sem_b)

            wait_s(38, rows_a, sem_sa)
            wait_s(39, rows_b, sem_sb)

        plsc.subcore_barrier()
        _per_tile_slabs(sid, lambda base, n: pltpu.sync_copy(
            acc.at[pl.ds(base, n)], out_ref.at[cid].at[pl.ds(base, n)]))

    return k(u, srcr, dstr)


def _dinv_block(deg_ref):
    deg = deg_ref[0, :, 0:1] + deg_ref[1, :, 0:1] + 1.0
    return lax.rsqrt(deg)


def _t1_body(x_ref, w_ref, deg_ref, out_ref):
    dinv = _dinv_block(deg_ref)
    t = jnp.dot(x_ref[...], w_ref[...], preferred_element_type=F32)
    out_ref[0, :, :] = t[:, :H] * dinv
    out_ref[1, :, :] = t[:, H:] * dinv


def _tmid_body(a_ref, deg_ref, b_ref, w_ref, out_ref):
    dinv = _dinv_block(deg_ref)
    act0 = jax.nn.relu(a_ref[0, :, :] * dinv + b_ref[0:1, :H])
    act1 = jax.nn.relu(a_ref[1, :, :] * dinv + b_ref[0:1, H:])
    t = (jnp.dot(act0, w_ref[:H, :], preferred_element_type=F32)
         + jnp.dot(act1, w_ref[H:, :], preferred_element_type=F32))
    out_ref[0, :, :] = t[:, :H] * dinv
    out_ref[1, :, :] = t[:, H:] * dinv


def _t5_body(a_ref, deg_ref, b_ref, batch_ref, lw1_ref, lb1_ref, lw2_ref,
             lb2_ref, lw3_ref, lb3_ref, out_ref, sums, cnt):
    i = pl.program_id(0)

    @pl.when(i == 0)
    def _():
        sums[...] = jnp.zeros_like(sums)
        cnt[...] = jnp.zeros_like(cnt)

    dinv = _dinv_block(deg_ref)
    h0 = jax.nn.relu(a_ref[0, :, :] * dinv + b_ref[0:1, :H])
    h1 = jax.nn.relu(a_ref[1, :, :] * dinv + b_ref[0:1, H:])
    iota_g = lax.broadcasted_iota(jnp.int32, (BN, NG), 1)
    oht = (batch_ref[...] == iota_g).astype(F32)
    dn = (((0,), (0,)), ((), ()))
    sums[:, :H] += lax.dot_general(oht, h0, dn, preferred_element_type=F32)
    sums[:, H:] += lax.dot_general(oht, h1, dn, preferred_element_type=F32)
    cnt[:, 0:1] += lax.dot_general(oht, jnp.ones((BN, 1), F32), dn,
                                   preferred_element_type=F32)

    @pl.when(i == pl.num_programs(0) - 1)
    def _():
        pooled = sums[...] / jnp.clip(cnt[:, 0:1], 1.0, None)
        g = jax.nn.relu(jnp.dot(pooled, lw1_ref[...],
                                preferred_element_type=F32) + lb1_ref[...])
        g = jax.nn.relu(jnp.dot(g, lw2_ref[...],
                                preferred_element_type=F32) + lb2_ref[...])
        out_ref[...] = jnp.dot(g, lw3_ref[...],
                               preferred_element_type=F32) + lb3_ref[...]


def _t1(x, W1, deg16):
    return pl.pallas_call(
        _t1_body,
        grid=(N // BN,),
        in_specs=[
            pl.BlockSpec((BN, D), lambda i: (i, 0)),
            pl.BlockSpec((D, D), lambda i: (0, 0)),
            pl.BlockSpec((NC, BN, 128), lambda i: (0, i, 0)),
        ],
        out_specs=pl.BlockSpec((NC, BN, H), lambda i: (0, i, 0)),
        out_shape=jax.ShapeDtypeStruct((NC, N, H), F32),
    )(x, W1, deg16)


def _tmid(a, deg16, b_row, W):
    return pl.pallas_call(
        _tmid_body,
        grid=(N // BN,),
        in_specs=[
            pl.BlockSpec((NC, BN, H), lambda i: (0, i, 0)),
            pl.BlockSpec((NC, BN, 128), lambda i: (0, i, 0)),
            pl.BlockSpec((1, D), lambda i: (0, 0)),
            pl.BlockSpec((D, D), lambda i: (0, 0)),
        ],
        out_specs=pl.BlockSpec((NC, BN, H), lambda i: (0, i, 0)),
        out_shape=jax.ShapeDtypeStruct((NC, N, H), F32),
    )(a, deg16, b_row, W)


def _t5(a, deg16, b_row, batch2d, lw1, lb1, lw2, lb2, lw3, lb3):
    return pl.pallas_call(
        _t5_body,
        grid=(N // BN,),
        in_specs=[
            pl.BlockSpec((NC, BN, H), lambda i: (0, i, 0)),
            pl.BlockSpec((NC, BN, 128), lambda i: (0, i, 0)),
            pl.BlockSpec((1, D), lambda i: (0, 0)),
            pl.BlockSpec((BN, 1), lambda i: (i, 0)),
            pl.BlockSpec((D, 128), lambda i: (0, 0)),
            pl.BlockSpec((1, 128), lambda i: (0, 0)),
            pl.BlockSpec((128, 64), lambda i: (0, 0)),
            pl.BlockSpec((1, 64), lambda i: (0, 0)),
            pl.BlockSpec((64, 1), lambda i: (0, 0)),
            pl.BlockSpec((1, 1), lambda i: (0, 0)),
        ],
        out_specs=pl.BlockSpec((NG, 1), lambda i: (0, 0)),
        out_shape=jax.ShapeDtypeStruct((NG, 1), F32),
        scratch_shapes=[pltpu.VMEM((NG, D), F32), pltpu.VMEM((NG, 128), F32)],
    )(a, deg16, b_row, batch2d, lw1, lb1, lw2, lb2, lw3, lb3)


def kernel(x, edge_index, batch, W1, b1, W2, b2, W3, b3, W4, b4,
           lw1, lb1, lw2, lb2, lw3, lb3):
    src = edge_index[0]
    dst = edge_index[1]
    npad = EPAD - E
    srcp = jnp.concatenate([src, jnp.zeros((npad,), jnp.int32)])
    dstp = jnp.concatenate([dst, jnp.full((npad,), N, jnp.int32)])
    srcr = srcp.reshape(NT, 80, 128)
    dstr = dstp.reshape(NT, 80, 128)
    dstd = dstp.reshape(NC, NT, 40, 128)
    zeros128 = jnp.zeros((NPAD, 128), F32)
    ones128 = jnp.ones((128, 128), F32)
    batch2d = batch.reshape(N, 1)
    b1r = b1.reshape(1, D)
    b2r = b2.reshape(1, D)
    b3r = b3.reshape(1, D)
    b4r = b4.reshape(1, D)

    deg16 = _sc_deg(dstd, zeros128, ones128)
    u = _t1(x, W1, deg16)
    for b_row, W in ((b1r, W2), (b2r, W3), (b3r, W4)):
        a = _sc_agg(u, srcr, dstr)
        u = _tmid(a, deg16, b_row, W)
    a = _sc_agg(u, srcr, dstr)
    return _t5(a, deg16, b4r, batch2d,
               lw1, lb1.reshape(1, 128), lw2, lb2.reshape(1, 64),
               lw3, lb3.reshape(1, 1))


# speedup vs baseline: 1.0770x; 1.0770x over previous
import functools

import jax
import jax.numpy as jnp
from jax import lax
from jax.experimental import pallas as pl
from jax.experimental.pallas import tpu as pltpu
from jax.experimental.pallas import tpu_sc as plsc

N = 10000
E = 160000
D = 256
H = 128
NG = 128
NC = 2
NT = 16
NPAD = 10240
EPAD = 163840
BN = 1000
F32 = jnp.float32

@functools.cache
def _mesh():
    return plsc.VectorSubcoreMesh(core_axis_name="c", subcore_axis_name="s")


def _per_tile_slabs(sid, fn):
    @pl.when(sid < NT - 1)
    def _():
        fn(sid * 640, 640)

    @pl.when(sid == NT - 1)
    def _():
        fn(9600, 400)


@functools.partial(jax.named_call, name="sc_deg")
def _sc_deg(dstd, zeros128, ones128):
    @functools.partial(
        pl.kernel,
        out_type=jax.ShapeDtypeStruct((NC, N, 128), F32),
        mesh=_mesh(),
        scratch_types=[
            pltpu.VMEM((40, 128), jnp.int32),
            pltpu.VMEM((128, 128), F32),
            pltpu.VMEM_SHARED((NPAD, 128), F32),
        ],
    )
    def k(dst_ref, z_ref, one_ref, out_ref, idxb, ones_v, acc):
        cid = lax.axis_index("c")
        sid = lax.axis_index("s")
        pltpu.sync_copy(z_ref.at[pl.ds(sid * 640, 640)],
                        acc.at[pl.ds(sid * 640, 640)])
        pltpu.sync_copy(dst_ref.at[cid].at[sid], idxb)
        pltpu.sync_copy(one_ref, ones_v)
        plsc.subcore_barrier()

        @pl.loop(0, 40)
        def _(j):
            pltpu.sync_copy(ones_v, acc.at[idxb.at[j]], add=True)

        plsc.subcore_barrier()
        _per_tile_slabs(sid, lambda base, n: pltpu.sync_copy(
            acc.at[pl.ds(base, n)], out_ref.at[cid].at[pl.ds(base, n)]))

    return k(dstd, zeros128, ones128)


@functools.partial(jax.named_call, name="sc_agg")
def _sc_agg(u, srcr, dstr):
    @functools.partial(
        pl.kernel,
        out_type=jax.ShapeDtypeStruct((NC, N, H), F32),
        mesh=_mesh(),
        scratch_types=[
            pltpu.VMEM((40, 128), jnp.int32),
            pltpu.VMEM((40, 128), jnp.int32),
            pltpu.VMEM((128, H), F32),
            pltpu.VMEM((128, H), F32),
            pltpu.VMEM_SHARED((NPAD, H), F32),
            pltpu.SemaphoreType.DMA,
            pltpu.SemaphoreType.DMA,
        ],
    )
    def k(u_ref, src_ref, dst_ref, out_ref, sbuf, dbuf, rows_a, rows_b,
          acc, sem_a, sem_b):
        cid = lax.axis_index("c")
        sid = lax.axis_index("s")
        _per_tile_slabs(sid, lambda base, n: pltpu.sync_copy(
            u_ref.at[cid].at[pl.ds(base, n)], acc.at[pl.ds(base, n)]))
        plsc.subcore_barrier()

        def gather(j, rows, sem):
            return pltpu.async_copy(u_ref.at[cid].at[sbuf.at[j]], rows, sem)

        for p in range(2):
            pltpu.sync_copy(src_ref.at[sid].at[pl.ds(p * 40, 40)], sbuf)
            pltpu.sync_copy(dst_ref.at[sid].at[pl.ds(p * 40, 40)], dbuf)
            gather(0, rows_a, sem_a)

            @pl.loop(0, 40, step=2)
            def _(j):
                gather(j + 1, rows_b, sem_b)
                pltpu.make_async_copy(u_ref.at[cid].at[sbuf.at[j]],
                                      rows_a, sem_a).wait()
                pltpu.sync_copy(rows_a, acc.at[dbuf.at[j]], add=True)

                @pl.when(j + 2 < 40)
                def _():
                    gather(j + 2, rows_a, sem_a)

                pltpu.make_async_copy(u_ref.at[cid].at[sbuf.at[j + 1]],
                                      rows_b, sem_b).wait()
                pltpu.sync_copy(rows_b, acc.at[dbuf.at[j + 1]], add=True)

        plsc.subcore_barrier()
        _per_tile_slabs(sid, lambda base, n: pltpu.sync_copy(
            acc.at[pl.ds(base, n)], out_ref.at[cid].at[pl.ds(base, n)]))

    return k(u, srcr, dstr)


def _dinv_block(deg_ref):
    deg = deg_ref[0, :, 0:1] + deg_ref[1, :, 0:1] + 1.0
    return lax.rsqrt(deg)


def _t1_body(x_ref, w_ref, deg_ref, out_ref):
    dinv = _dinv_block(deg_ref)
    t = jnp.dot(x_ref[...], w_ref[...], preferred_element_type=F32)
    out_ref[0, :, :] = t[:, :H] * dinv
    out_ref[1, :, :] = t[:, H:] * dinv


def _tmid_body(a_ref, deg_ref, b_ref, w_ref, out_ref):
    dinv = _dinv_block(deg_ref)
    act0 = jax.nn.relu(a_ref[0, :, :] * dinv + b_ref[0:1, :H])
    act1 = jax.nn.relu(a_ref[1, :, :] * dinv + b_ref[0:1, H:])
    t = (jnp.dot(act0, w_ref[:H, :], preferred_element_type=F32)
         + jnp.dot(act1, w_ref[H:, :], preferred_element_type=F32))
    out_ref[0, :, :] = t[:, :H] * dinv
    out_ref[1, :, :] = t[:, H:] * dinv


def _t5_body(a_ref, deg_ref, b_ref, batch_ref, lw1_ref, lb1_ref, lw2_ref,
             lb2_ref, lw3_ref, lb3_ref, out_ref, sums, cnt):
    i = pl.program_id(0)

    @pl.when(i == 0)
    def _():
        sums[...] = jnp.zeros_like(sums)
        cnt[...] = jnp.zeros_like(cnt)

    dinv = _dinv_block(deg_ref)
    h0 = jax.nn.relu(a_ref[0, :, :] * dinv + b_ref[0:1, :H])
    h1 = jax.nn.relu(a_ref[1, :, :] * dinv + b_ref[0:1, H:])
    iota_g = lax.broadcasted_iota(jnp.int32, (BN, NG), 1)
    oht = (batch_ref[...] == iota_g).astype(F32)
    dn = (((0,), (0,)), ((), ()))
    sums[:, :H] += lax.dot_general(oht, h0, dn, preferred_element_type=F32)
    sums[:, H:] += lax.dot_general(oht, h1, dn, preferred_element_type=F32)
    cnt[:, 0:1] += lax.dot_general(oht, jnp.ones((BN, 1), F32), dn,
                                   preferred_element_type=F32)

    @pl.when(i == pl.num_programs(0) - 1)
    def _():
        pooled = sums[...] / jnp.clip(cnt[:, 0:1], 1.0, None)
        g = jax.nn.relu(jnp.dot(pooled, lw1_ref[...],
                                preferred_element_type=F32) + lb1_ref[...])
        g = jax.nn.relu(jnp.dot(g, lw2_ref[...],
                                preferred_element_type=F32) + lb2_ref[...])
        out_ref[...] = jnp.dot(g, lw3_ref[...],
                               preferred_element_type=F32) + lb3_ref[...]


def _t1(x, W1, deg16):
    return pl.pallas_call(
        _t1_body,
        grid=(N // BN,),
        in_specs=[
            pl.BlockSpec((BN, D), lambda i: (i, 0)),
            pl.BlockSpec((D, D), lambda i: (0, 0)),
            pl.BlockSpec((NC, BN, 128), lambda i: (0, i, 0)),
        ],
        out_specs=pl.BlockSpec((NC, BN, H), lambda i: (0, i, 0)),
        out_shape=jax.ShapeDtypeStruct((NC, N, H), F32),
    )(x, W1, deg16)


def _tmid(a, deg16, b_row, W):
    return pl.pallas_call(
        _tmid_body,
        grid=(N // BN,),
        in_specs=[
            pl.BlockSpec((NC, BN, H), lambda i: (0, i, 0)),
            pl.BlockSpec((NC, BN, 128), lambda i: (0, i, 0)),
            pl.BlockSpec((1, D), lambda i: (0, 0)),
            pl.BlockSpec((D, D), lambda i: (0, 0)),
        ],
        out_specs=pl.BlockSpec((NC, BN, H), lambda i: (0, i, 0)),
        out_shape=jax.ShapeDtypeStruct((NC, N, H), F32),
    )(a, deg16, b_row, W)


def _t5(a, deg16, b_row, batch2d, lw1, lb1, lw2, lb2, lw3, lb3):
    return pl.pallas_call(
        _t5_body,
        grid=(N // BN,),
        in_specs=[
            pl.BlockSpec((NC, BN, H), lambda i: (0, i, 0)),
            pl.BlockSpec((NC, BN, 128), lambda i: (0, i, 0)),
            pl.BlockSpec((1, D), lambda i: (0, 0)),
            pl.BlockSpec((BN, 1), lambda i: (i, 0)),
            pl.BlockSpec((D, 128), lambda i: (0, 0)),
            pl.BlockSpec((1, 128), lambda i: (0, 0)),
            pl.BlockSpec((128, 64), lambda i: (0, 0)),
            pl.BlockSpec((1, 64), lambda i: (0, 0)),
            pl.BlockSpec((64, 1), lambda i: (0, 0)),
            pl.BlockSpec((1, 1), lambda i: (0, 0)),
        ],
        out_specs=pl.BlockSpec((NG, 1), lambda i: (0, 0)),
        out_shape=jax.ShapeDtypeStruct((NG, 1), F32),
        scratch_shapes=[pltpu.VMEM((NG, D), F32), pltpu.VMEM((NG, 128), F32)],
    )(a, deg16, b_row, batch2d, lw1, lb1, lw2, lb2, lw3, lb3)


def kernel(x, edge_index, batch, W1, b1, W2, b2, W3, b3, W4, b4,
           lw1, lb1, lw2, lb2, lw3, lb3):
    src = edge_index[0]
    dst = edge_index[1]
    npad = EPAD - E
    srcp = jnp.concatenate([src, jnp.zeros((npad,), jnp.int32)])
    dstp = jnp.concatenate([dst, jnp.full((npad,), N, jnp.int32)])
    srcr = srcp.reshape(NT, 80, 128)
    dstr = dstp.reshape(NT, 80, 128)
    dstd = dstp.reshape(NC, NT, 40, 128)
    zeros128 = jnp.zeros((NPAD, 128), F32)
    ones128 = jnp.ones((128, 128), F32)
    batch2d = batch.reshape(N, 1)
    b1r = b1.reshape(1, D)
    b2r = b2.reshape(1, D)
    b3r = b3.reshape(1, D)
    b4r = b4.reshape(1, D)

    deg16 = _sc_deg(dstd, zeros128, ones128)
    u = _t1(x, W1, deg16)
    for b_row, W in ((b1r, W2), (b2r, W3), (b3r, W4)):
        a = _sc_agg(u, srcr, dstr)
        u = _tmid(a, deg16, b_row, W)
    a = _sc_agg(u, srcr, dstr)
    return _t5(a, deg16, b4r, batch2d,
               lw1, lb1.reshape(1, 128), lw2, lb2.reshape(1, 64),
               lw3, lb3.reshape(1, 1))
